# baseline (speedup 1.0000x reference)
import functools

import jax
import jax.numpy as jnp
import jax.scipy.linalg
from jax import lax
from jax.experimental import pallas as pl
from jax.experimental.pallas import tpu as pltpu
from jax.experimental.pallas import tpu_sc as plsc

N = 10000
E = 320000
IN_DIM = 128
HID_DIM = 32
OUT_DIM = 64

NC = 2
NS = 16
NW = NC * NS
LANES = 16

B = 128
NBLKT = E // B
NBLK = 78
TAILB = NBLKT - NW * NBLK // 1
NPADD = 10240
RPT = N // NS
ZR = 125
NBUF = 8
PK = N // 4
F32 = jnp.float32


def _worker_id():
    return lax.axis_index("c") * NS + lax.axis_index("s")


def _make_deg_kernel():
    mesh = plsc.VectorSubcoreMesh(core_axis_name="c", subcore_axis_name="s")

    @functools.partial(
        pl.kernel,
        mesh=mesh,
        out_type=jax.ShapeDtypeStruct((NW, NPADD), F32),
        scratch_types=[
            pltpu.VMEM((NBLK, B), jnp.int32),
            pltpu.VMEM((LANES,), jnp.int32),
            pltpu.VMEM((NPADD,), F32),
        ],
        compiler_params=pltpu.CompilerParams(needs_layout_passes=False,
                                             use_tc_tiling_on_sc=False),
    )
    def deg_kernel(ei_hbm, out_hbm, dst_v, tdst_v, deg_v):
        wid = _worker_id()
        b0 = wid * NBLK
        pltpu.sync_copy(ei_hbm.at[1, pl.ds(b0, NBLK)], dst_v)
        pltpu.sync_copy(
            ei_hbm.at[1, NW * NBLK + wid // 8, pl.ds((wid % 8) * LANES, LANES)],
            tdst_v)

        zeros16 = jnp.zeros((LANES,), F32)
        ones16 = jnp.full((LANES,), 1.0, F32)

        @pl.loop(0, NPADD // LANES)
        def _zero(i):
            deg_v[pl.ds(i * LANES, LANES)] = zeros16

        @pl.loop(0, NBLK)
        def _acc(jb):
            for k in range(B // LANES):
                idx = dst_v[jb, pl.ds(k * LANES, LANES)]
                plsc.addupdate_scatter(deg_v, [idx], ones16)

        plsc.addupdate_scatter(deg_v, [tdst_v[...]], ones16)
        pltpu.sync_copy(deg_v, out_hbm.at[wid])

    return deg_kernel


def _make_agg_kernel(F):
    mesh = plsc.VectorSubcoreMesh(core_axis_name="c", subcore_axis_name="s")
    NMAIN = (NBLK // NBUF - 1) * NBUF

    @functools.partial(
        pl.kernel,
        mesh=mesh,
        out_type=jax.ShapeDtypeStruct((NC, N, F), F32),
        scratch_types=[
            pltpu.VMEM((NBLK, B), jnp.int32),
            pltpu.VMEM((NBLK, B), jnp.int32),
            pltpu.VMEM((LANES,), jnp.int32),
            pltpu.VMEM((LANES,), jnp.int32),
            pltpu.VMEM((RPT // ZR, ZR), jnp.int32),
            [pltpu.VMEM((B, F), F32)] * NBUF,
            pltpu.VMEM((LANES, F), F32),
            pltpu.VMEM((ZR, F), F32),
            pltpu.VMEM_SHARED((N, F), F32),
            [pltpu.SemaphoreType.DMA] * NBUF,
            [pltpu.SemaphoreType.DMA] * NBUF,
        ],
        compiler_params=pltpu.CompilerParams(use_tc_tiling_on_sc=False),
    )
    def agg_kernel(g_hbm, ei_hbm, ident_hbm, out_hbm,
                   src_v, dst_v, tsrc_v, tdst_v, ident_v, rows, trows, zbuf,
                   acc, sems, ssems):
        c = lax.axis_index("c")
        s = lax.axis_index("s")
        wid = c * NS + s
        base = s * RPT
        b0 = wid * NBLK

        pltpu.sync_copy(ei_hbm.at[0, pl.ds(b0, NBLK)], src_v)
        pltpu.sync_copy(ei_hbm.at[1, pl.ds(b0, NBLK)], dst_v)
        pltpu.sync_copy(
            ei_hbm.at[0, NW * NBLK + wid // 8, pl.ds((wid % 8) * LANES, LANES)],
            tsrc_v)
        pltpu.sync_copy(
            ei_hbm.at[1, NW * NBLK + wid // 8, pl.ds((wid % 8) * LANES, LANES)],
            tdst_v)
        pltpu.sync_copy(ident_hbm.at[pl.ds(s * (RPT // ZR), RPT // ZR)],
                        ident_v)

        zeros16 = jnp.zeros((LANES,), F32)

        @pl.loop(0, ZR)
        def _zero(i):
            for jj in range(F // LANES):
                zbuf[i, pl.ds(jj * LANES, LANES)] = zeros16

        for k in range(RPT // ZR):
            pltpu.sync_copy(zbuf, acc.at[pl.ds(base + k * ZR, ZR)])

        plsc.subcore_barrier()

        D = NBUF // 2
        MMAIN = (NBLK - 2 * D) // NBUF * NBUF

        def _gfire(blk, bb):
            pltpu.async_copy(g_hbm.at[src_v.at[blk]], rows[bb], sems[bb])

        def _gwait(blk, bb):
            pltpu.make_async_copy(g_hbm.at[src_v.at[blk]], rows[bb],
                                  sems[bb]).wait()

        def _sfire(blk, bb):
            pltpu.async_copy(rows[bb], acc.at[dst_v.at[blk]], ssems[bb],
                             add=True)

        def _swait(blk, bb):
            pltpu.make_async_copy(rows[bb], acc.at[dst_v.at[blk]],
                                  ssems[bb]).wait()

        for blk in range(D):
            _gfire(blk, blk % NBUF)

        for blk in range(D):
            _gwait(blk, blk % NBUF)
            _sfire(blk, blk % NBUF)
            _gfire(blk + D, (blk + D) % NBUF)

        @pl.loop(0, MMAIN // NBUF)
        def _edges(j):
            for b in range(NBUF):
                blk = D + j * NBUF + b
                bb = (D + b) % NBUF
                _gwait(blk, bb)
                _sfire(blk, bb)
                _swait(blk - D, b)
                _gfire(blk + D, b)

        for sblk in range(D + MMAIN, NBLK - D):
            _gwait(sblk, sblk % NBUF)
            _sfire(sblk, sblk % NBUF)
            _swait(sblk - D, (sblk - D) % NBUF)
            _gfire(sblk + D, (sblk - D) % NBUF)

        for sblk in range(NBLK - D, NBLK):
            _gwait(sblk, sblk % NBUF)
            _sfire(sblk, sblk % NBUF)
            _swait(sblk - D, (sblk - D) % NBUF)

        for sblk in range(NBLK - D, NBLK):
            _swait(sblk, sblk % NBUF)

        pltpu.async_copy(g_hbm.at[tsrc_v], trows, sems[0]).wait()
        pltpu.sync_copy(trows, acc.at[tdst_v], add=True)

        for k in range(RPT // ZR):
            @pl.when(c == (0 if k < 3 else 1))
            def _self_add():
                pltpu.sync_copy(g_hbm.at[pl.ds(base + k * ZR, ZR)], zbuf)
                pltpu.sync_copy(zbuf, acc.at[ident_v.at[k]], add=True)

        plsc.subcore_barrier()

        for k in range(RPT // ZR):
            off = base + k * ZR
            pltpu.sync_copy(acc.at[pl.ds(off, ZR)], out_hbm.at[c, pl.ds(off, ZR)])

    return agg_kernel


_deg_kernel = _make_deg_kernel()
_agg_hid = _make_agg_kernel(HID_DIM)


def _tc_a1_body(deg_ref, dinv_ref):
    deg = jnp.sum(deg_ref[...], axis=0) + 1.0
    dinv_ref[...] = lax.rsqrt(deg)


def _tc_a2_body(x_ref, w_ref, dinvn_ref, g_ref):
    h = jnp.dot(x_ref[...], w_ref[...], preferred_element_type=F32)
    g_ref[...] = h * dinvn_ref[...]


def _tc_b_body(agg_ref, dinvp_ref, b_ref, q_ref):
    dinv = dinvp_ref[...]
    out1 = jnp.maximum(dinv * (agg_ref[0] + agg_ref[1]) + b_ref[...], 0.0)
    q_ref[...] = out1 * dinv


def _tc_c_body(agg_ref, dinvp_ref, w_ref, b_ref, o_ref):
    pre = dinvp_ref[...] * (agg_ref[0] + agg_ref[1])
    h2 = jnp.dot(pre, w_ref[...], preferred_element_type=F32) + b_ref[...]
    out2 = jnp.maximum(h2, 0.0)
    ssum = jnp.sum(out2, axis=0, keepdims=True)
    o_ref[...] = (ssum[:, 0:64] + ssum[:, 64:128]
                  + ssum[:, 128:192] + ssum[:, 192:256]) * (1.0 / N)


def kernel(x, edge_index, W1, b1, W2, b2):
    ei = edge_index.astype(jnp.int32).reshape(2, NBLKT, B)

    ident = jnp.arange(N, dtype=jnp.int32).reshape(NS * (RPT // ZR), ZR)

    deg_p = _deg_kernel(ei)

    dinv_c = pl.pallas_call(
        _tc_a1_body,
        out_shape=jax.ShapeDtypeStruct((NPADD // 128, 128), F32),
    )(deg_p.reshape(NW, NPADD // 128, 128))

    dinv_flat = dinv_c.reshape(NPADD)[:N, None]
    dinv_n = jnp.broadcast_to(dinv_flat, (N, HID_DIM))
    dinv_p = jnp.broadcast_to(dinv_flat, (N, HID_DIM)).reshape(PK, 128)

    g1 = pl.pallas_call(
        _tc_a2_body,
        out_shape=jax.ShapeDtypeStruct((N, HID_DIM), F32),
    )(x, W1, dinv_n)

    agg1 = _agg_hid(g1, ei, ident)

    qp = pl.pallas_call(
        _tc_b_body,
        out_shape=jax.ShapeDtypeStruct((PK, 128), F32),
    )(agg1.reshape(NC, PK, 128), dinv_p, jnp.tile(b1, 4).reshape(1, 128))

    agg2 = _agg_hid(qp.reshape(N, HID_DIM), ei, ident)

    w2big = jax.scipy.linalg.block_diag(W2, W2, W2, W2)
    out = pl.pallas_call(
        _tc_c_body,
        out_shape=jax.ShapeDtypeStruct((1, OUT_DIM), F32),
    )(agg2.reshape(NC, PK, 128), dinv_p, w2big,
      jnp.tile(b2, 4).reshape(1, 256))

    return out.reshape(OUT_DIM)


# speedup vs baseline: 1.0628x; 1.0628x over previous
import functools

import jax
import jax.numpy as jnp
import jax.scipy.linalg
from jax import lax
from jax.experimental import pallas as pl
from jax.experimental.pallas import tpu as pltpu
from jax.experimental.pallas import tpu_sc as plsc

N = 10000
E = 320000
IN_DIM = 128
HID_DIM = 32
OUT_DIM = 64

NC = 2
NS = 16
NW = NC * NS
LANES = 16

B = 128
NBLKT = E // B
NBLK = 78
TAILB = NBLKT - NW * NBLK // 1
NPADD = 10240
RPT = N // NS
ZR = 125
NBUF = 8
PK = N // 4
F32 = jnp.float32


def _worker_id():
    return lax.axis_index("c") * NS + lax.axis_index("s")


def _make_deg_kernel():
    mesh = plsc.VectorSubcoreMesh(core_axis_name="c", subcore_axis_name="s")

    @functools.partial(
        pl.kernel,
        mesh=mesh,
        out_type=jax.ShapeDtypeStruct((NW, NPADD), F32),
        scratch_types=[
            pltpu.VMEM((NBLK, B), jnp.int32),
            pltpu.VMEM((LANES,), jnp.int32),
            pltpu.VMEM((NPADD,), F32),
        ],
        compiler_params=pltpu.CompilerParams(needs_layout_passes=False,
                                             use_tc_tiling_on_sc=False),
    )
    def deg_kernel(ei_hbm, out_hbm, dst_v, tdst_v, deg_v):
        wid = _worker_id()
        b0 = wid * NBLK
        pltpu.sync_copy(ei_hbm.at[1, pl.ds(b0, NBLK)], dst_v)
        pltpu.sync_copy(
            ei_hbm.at[1, NW * NBLK + wid // 8, pl.ds((wid % 8) * LANES, LANES)],
            tdst_v)

        zeros16 = jnp.zeros((LANES,), F32)
        ones16 = jnp.full((LANES,), 1.0, F32)

        @pl.loop(0, NPADD // LANES)
        def _zero(i):
            deg_v[pl.ds(i * LANES, LANES)] = zeros16

        @pl.loop(0, NBLK)
        def _acc(jb):
            for k in range(B // LANES):
                idx = dst_v[jb, pl.ds(k * LANES, LANES)]
                plsc.addupdate_scatter(deg_v, [idx], ones16)

        plsc.addupdate_scatter(deg_v, [tdst_v[...]], ones16)
        pltpu.sync_copy(deg_v, out_hbm.at[wid])

    return deg_kernel


def _make_agg_kernel(F):
    mesh = plsc.VectorSubcoreMesh(core_axis_name="c", subcore_axis_name="s")
    NMAIN = (NBLK // NBUF - 1) * NBUF

    @functools.partial(
        pl.kernel,
        mesh=mesh,
        out_type=jax.ShapeDtypeStruct((NC, N, F), F32),
        scratch_types=[
            pltpu.VMEM((NBLK, B), jnp.int32),
            pltpu.VMEM((NBLK, B), jnp.int32),
            pltpu.VMEM((LANES,), jnp.int32),
            pltpu.VMEM((LANES,), jnp.int32),
            pltpu.VMEM((RPT // ZR, ZR), jnp.int32),
            [pltpu.VMEM((B, F), F32)] * NBUF,
            pltpu.VMEM((LANES, F), F32),
            pltpu.VMEM((ZR, F), F32),
            pltpu.VMEM_SHARED((N, F), F32),
            [pltpu.SemaphoreType.DMA] * NBUF,
        ],
        compiler_params=pltpu.CompilerParams(use_tc_tiling_on_sc=False),
    )
    def agg_kernel(g_hbm, ei_hbm, ident_hbm, out_hbm,
                   src_v, dst_v, tsrc_v, tdst_v, ident_v, rows, trows, zbuf,
                   acc, sems):
        c = lax.axis_index("c")
        s = lax.axis_index("s")
        wid = c * NS + s
        base = s * RPT
        b0 = wid * NBLK

        pltpu.sync_copy(ei_hbm.at[0, pl.ds(b0, NBLK)], src_v)
        pltpu.sync_copy(ei_hbm.at[1, pl.ds(b0, NBLK)], dst_v)
        pltpu.sync_copy(
            ei_hbm.at[0, NW * NBLK + wid // 8, pl.ds((wid % 8) * LANES, LANES)],
            tsrc_v)
        pltpu.sync_copy(
            ei_hbm.at[1, NW * NBLK + wid // 8, pl.ds((wid % 8) * LANES, LANES)],
            tdst_v)
        pltpu.sync_copy(ident_hbm.at[pl.ds(s * (RPT // ZR), RPT // ZR)],
                        ident_v)

        zeros16 = jnp.zeros((LANES,), F32)

        @pl.loop(0, ZR)
        def _zero(i):
            for jj in range(F // LANES):
                zbuf[i, pl.ds(jj * LANES, LANES)] = zeros16

        for k in range(RPT // ZR):
            pltpu.sync_copy(zbuf, acc.at[pl.ds(base + k * ZR, ZR)])

        plsc.subcore_barrier()

        for b in range(NBUF):
            pltpu.async_copy(g_hbm.at[src_v.at[b]], rows[b], sems[b])

        @pl.loop(0, NMAIN // NBUF)
        def _edges(j):
            for b in range(NBUF):
                blk = j * NBUF + b
                pltpu.make_async_copy(g_hbm.at[src_v.at[blk]], rows[b],
                                      sems[b]).wait()
                pltpu.sync_copy(rows[b], acc.at[dst_v.at[blk]], add=True)
                pltpu.async_copy(g_hbm.at[src_v.at[blk + NBUF]], rows[b],
                                 sems[b])

        for blk in range(NMAIN, NBLK):
            b = blk % NBUF
            pltpu.make_async_copy(g_hbm.at[src_v.at[blk]], rows[b],
                                  sems[b]).wait()
            pltpu.sync_copy(rows[b], acc.at[dst_v.at[blk]], add=True)
            if blk + NBUF < NBLK:
                pltpu.async_copy(g_hbm.at[src_v.at[blk + NBUF]], rows[b],
                                 sems[b])

        pltpu.async_copy(g_hbm.at[tsrc_v], trows, sems[0]).wait()
        pltpu.sync_copy(trows, acc.at[tdst_v], add=True)

        for k in range(RPT // ZR):
            @pl.when(c == (0 if k < 3 else 1))
            def _self_add():
                pltpu.sync_copy(g_hbm.at[pl.ds(base + k * ZR, ZR)], zbuf)
                pltpu.sync_copy(zbuf, acc.at[ident_v.at[k]], add=True)

        plsc.subcore_barrier()

        for k in range(RPT // ZR):
            off = base + k * ZR
            pltpu.sync_copy(acc.at[pl.ds(off, ZR)], out_hbm.at[c, pl.ds(off, ZR)])

    return agg_kernel


_deg_kernel = _make_deg_kernel()
_agg_hid = _make_agg_kernel(HID_DIM)


def _tc_a1_body(deg_ref, dinv_ref):
    deg = jnp.sum(deg_ref[...], axis=0) + 1.0
    dinv_ref[...] = lax.rsqrt(deg)


def _tc_a2_body(x_ref, w_ref, dinvn_ref, g_ref):
    h = jnp.dot(x_ref[...], w_ref[...], preferred_element_type=F32)
    g_ref[...] = h * dinvn_ref[...]


def _tc_b_body(agg_ref, dinvp_ref, b_ref, q_ref):
    dinv = dinvp_ref[...]
    out1 = jnp.maximum(dinv * (agg_ref[0] + agg_ref[1]) + b_ref[...], 0.0)
    q_ref[...] = out1 * dinv


def _tc_c_body(agg_ref, dinvp_ref, w_ref, b_ref, o_ref):
    pre = dinvp_ref[...] * (agg_ref[0] + agg_ref[1])
    h2 = jnp.dot(pre, w_ref[...], preferred_element_type=F32) + b_ref[...]
    out2 = jnp.maximum(h2, 0.0)
    ssum = jnp.sum(out2, axis=0, keepdims=True)
    o_ref[...] = (ssum[:, 0:64] + ssum[:, 64:128]
                  + ssum[:, 128:192] + ssum[:, 192:256]) * (1.0 / N)


def kernel(x, edge_index, W1, b1, W2, b2):
    ei = edge_index.astype(jnp.int32).reshape(2, NBLKT, B)

    ident = jnp.arange(N, dtype=jnp.int32).reshape(NS * (RPT // ZR), ZR)

    deg_p = _deg_kernel(ei)

    dinv_c = pl.pallas_call(
        _tc_a1_body,
        out_shape=jax.ShapeDtypeStruct((NPADD // 128, 128), F32),
    )(deg_p.reshape(NW, NPADD // 128, 128))

    dinv_flat = dinv_c.reshape(NPADD)[:N, None]
    dinv_n = jnp.broadcast_to(dinv_flat, (N, HID_DIM))
    dinv_p = jnp.broadcast_to(dinv_flat, (N, HID_DIM)).reshape(PK, 128)

    g1 = pl.pallas_call(
        _tc_a2_body,
        out_shape=jax.ShapeDtypeStruct((N, HID_DIM), F32),
    )(x, W1, dinv_n)

    agg1 = _agg_hid(g1, ei, ident)

    qp = pl.pallas_call(
        _tc_b_body,
        out_shape=jax.ShapeDtypeStruct((PK, 128), F32),
    )(agg1.reshape(NC, PK, 128), dinv_p, jnp.tile(b1, 4).reshape(1, 128))

    agg2 = _agg_hid(qp.reshape(N, HID_DIM), ei, ident)

    w2big = jax.scipy.linalg.block_diag(W2, W2, W2, W2)
    out = pl.pallas_call(
        _tc_c_body,
        out_shape=jax.ShapeDtypeStruct((1, OUT_DIM), F32),
    )(agg2.reshape(NC, PK, 128), dinv_p, w2big,
      jnp.tile(b2, 4).reshape(1, 256))

    return out.reshape(OUT_DIM)
